# baseline (speedup 1.0000x reference)
import functools

import jax
import jax.numpy as jnp
from jax import lax
from jax.experimental import pallas as pl
from jax.experimental.pallas import tpu as pltpu
from jax.experimental.pallas import tpu_sc as plsc

N = 10000
E = 320000
D = 128
DE = 16
NC = 2
NS = 16
NW = NC * NS
CH = 128
EPT = E // NW
NCH = 80
KB = 2
NB = NCH // KB
HB = KB * CH
ZR = 64
EPT_PAD = NCH * CH
EPAD = EPT_PAD * NW
TRASH = N
RPT = 632
NP = NS * RPT
ZCH = [(0, 128), (128, 128), (256, 128), (384, 128), (512, 120)]

_mesh = plsc.VectorSubcoreMesh(core_axis_name="c", subcore_axis_name="s",
                               num_cores=NC, num_subcores=NS)


def _zero_rows(buf, nrow, ncol):
    zeros16 = jnp.zeros((16,), jnp.float32)

    def body(r, _):
        for c in range(ncol // 16):
            buf[r, pl.ds(c * 16, 16)] = zeros16
        return 0

    lax.fori_loop(0, nrow, body, 0)


@functools.partial(
    pl.kernel,
    out_type=jax.ShapeDtypeStruct((NC, NP, D), jnp.float32),
    mesh=_mesh,
    scratch_types=[
        pltpu.VMEM((NCH, CH), jnp.int32),
        pltpu.VMEM((CH, D), jnp.float32),
        pltpu.VMEM_SHARED((NP, D), jnp.float32),
    ],
)
def _sc_deg(dstp, out, dst_v, val_v, acc_sh):
    cid = lax.axis_index("c")
    sid = lax.axis_index("s")
    wid = sid * NC + cid

    _zero_rows(val_v, CH, D)
    for off, nn in ZCH:
        pltpu.sync_copy(val_v.at[pl.ds(0, nn)],
                        acc_sh.at[pl.ds(sid * RPT + off, nn)])
    pltpu.sync_copy(dstp.at[wid], dst_v)
    plsc.subcore_barrier()

    ones16 = jnp.ones((16,), jnp.float32)

    def fill(r, _):
        for c in range(D // 16):
            val_v[r, pl.ds(c * 16, 16)] = ones16
        return 0

    lax.fori_loop(0, CH, fill, 0)

    def chunk(j, _):
        pltpu.sync_copy(val_v, acc_sh.at[dst_v.at[j]], add=True)
        return 0

    lax.fori_loop(0, NCH, chunk, 0)
    plsc.subcore_barrier()

    for off, nn in ZCH:
        sl = pl.ds(sid * RPT + off, nn)
        vv = val_v.at[pl.ds(0, nn)]
        pltpu.sync_copy(acc_sh.at[sl], vv)
        pltpu.sync_copy(vv, out.at[cid, sl])


@functools.partial(
    pl.kernel,
    out_type=jax.ShapeDtypeStruct((NC, NP, D), jnp.float32),
    mesh=_mesh,
    scratch_types=[
        [pltpu.VMEM((CH,), jnp.int32)] * 3,
        [pltpu.VMEM((CH,), jnp.int32)] * 3,
        [pltpu.VMEM((CH, D), jnp.float32)] * 3,
        [pltpu.SemaphoreType.DMA] * 3,
        [pltpu.SemaphoreType.DMA] * 3,
        pltpu.VMEM_SHARED((NP, D), jnp.float32),
    ],
)
def _sc_agg(table, srcp, dstp, out, sidx, didx, rows, si, sg, acc_sh):
    cid = lax.axis_index("c")
    sid = lax.axis_index("s")
    wid = sid * NC + cid

    _zero_rows(rows[0], CH, D)
    for off, nn in ZCH:
        pltpu.sync_copy(rows[0].at[pl.ds(0, nn)],
                        acc_sh.at[pl.ds(sid * RPT + off, nn)])
    plsc.subcore_barrier()

    pltpu.sync_copy(srcp.at[wid, 0], sidx[0])
    pltpu.sync_copy(dstp.at[wid, 0], didx[0])
    for k in (1, 2):
        pltpu.async_copy(srcp.at[wid, k], sidx[k], si[k])
        pltpu.async_copy(dstp.at[wid, k], didx[k], si[k])
    pltpu.async_copy(table.at[sidx[0]], rows[0], sg[0])

    def chunk(j, _):
        r = lax.rem(j, 3)
        for k in range(3):
            nxt = (k + 1) % 3

            @pl.when(r == k)
            def _():
                @pl.when(j + 1 < NCH)
                def _():
                    pltpu.make_async_copy(srcp.at[wid, j + 1], sidx[nxt], si[nxt]).wait()
                    pltpu.make_async_copy(dstp.at[wid, j + 1], didx[nxt], si[nxt]).wait()
                    pltpu.async_copy(table.at[sidx[nxt]], rows[nxt], sg[nxt])

                pltpu.make_async_copy(table.at[sidx[k]], rows[k], sg[k]).wait()
                pltpu.sync_copy(rows[k], acc_sh.at[didx[k]], add=True)

                @pl.when(j + 3 < NCH)
                def _():
                    pltpu.async_copy(srcp.at[wid, j + 3], sidx[k], si[k])
                    pltpu.async_copy(dstp.at[wid, j + 3], didx[k], si[k])
        return 0

    lax.fori_loop(0, NCH, chunk, 0)
    plsc.subcore_barrier()

    for off, nn in ZCH:
        sl = pl.ds(sid * RPT + off, nn)
        vv = rows[0].at[pl.ds(0, nn)]
        pltpu.sync_copy(acc_sh.at[sl], vv)
        pltpu.sync_copy(vv, out.at[cid, sl])


@functools.partial(
    pl.kernel,
    out_type=jax.ShapeDtypeStruct((EPAD, D), jnp.float32),
    mesh=_mesh,
    scratch_types=[
        pltpu.VMEM((NCH, CH), jnp.int32),
        pltpu.VMEM((NCH, CH), jnp.int32),
        pltpu.VMEM((CH, D), jnp.float32),
        pltpu.VMEM((CH, D), jnp.float32),
        pltpu.VMEM((CH, D), jnp.float32),
        pltpu.VMEM((CH, D), jnp.float32),
        pltpu.SemaphoreType.DMA,
        pltpu.SemaphoreType.DMA,
        pltpu.SemaphoreType.DMA,
        pltpu.SemaphoreType.DMA,
    ],
)
def _sc_pair(hA, hB, srcp, dstp, out, src_v, dst_v, ra0, rb0, ra1, rb1,
             semA0, semB0, semA1, semB1):
    cid = lax.axis_index("c")
    sid = lax.axis_index("s")
    wid = sid * NC + cid

    pltpu.sync_copy(srcp.at[wid], src_v)
    pltpu.sync_copy(dstp.at[wid], dst_v)

    def add_store(ra, rb, j):
        def add_row(r, _):
            for c in range(D // 16):
                sl = pl.ds(c * 16, 16)
                ra[r, sl] = ra[r, sl] + rb[r, sl]
            return 0

        lax.fori_loop(0, CH, add_row, 0)
        pltpu.sync_copy(ra, out.at[pl.ds((wid * NCH + j) * CH, CH)])

    pltpu.async_copy(hA.at[src_v.at[0]], ra0, semA0)
    pltpu.async_copy(hB.at[dst_v.at[0]], rb0, semB0)

    def pair2(p, _):
        j0 = 2 * p
        j1 = j0 + 1
        pltpu.make_async_copy(hA.at[src_v.at[j0]], ra0, semA0).wait()
        pltpu.make_async_copy(hB.at[dst_v.at[j0]], rb0, semB0).wait()
        pltpu.async_copy(hA.at[src_v.at[j1]], ra1, semA1)
        pltpu.async_copy(hB.at[dst_v.at[j1]], rb1, semB1)
        add_store(ra0, rb0, j0)
        pltpu.make_async_copy(hA.at[src_v.at[j1]], ra1, semA1).wait()
        pltpu.make_async_copy(hB.at[dst_v.at[j1]], rb1, semB1).wait()

        @pl.when(p + 1 < NCH // 2)
        def _():
            pltpu.async_copy(hA.at[src_v.at[j0 + 2]], ra0, semA0)
            pltpu.async_copy(hB.at[dst_v.at[j0 + 2]], rb0, semB0)

        add_store(ra1, rb1, j1)
        return 0

    lax.fori_loop(0, NCH // 2, pair2, 0)


_NB = 10
_BR = N // _NB


def _prep_body(x_ref, w_ref, degp_ref, dinv_ref, hs_ref):
    deg = degp_ref[0, :, 0:1] + degp_ref[1, :, 0:1] + 1.0
    dinv = lax.rsqrt(deg)
    xv = x_ref[...]
    xv = jnp.where(jnp.isnan(xv), 0.0, xv)
    dinv_ref[...] = dinv
    hs_ref[...] = jnp.dot(xv, w_ref[...], preferred_element_type=jnp.float32) * dinv


def _tc_prep(x, W0, degp):
    return pl.pallas_call(
        _prep_body,
        grid=(_NB,),
        in_specs=[
            pl.BlockSpec((_BR, D), lambda i: (i, 0)),
            pl.BlockSpec((D, D), lambda i: (0, 0)),
            pl.BlockSpec((NC, _BR, D), lambda i: (0, i, 0)),
        ],
        out_specs=[
            pl.BlockSpec((_BR, 1), lambda i: (i, 0)),
            pl.BlockSpec((_BR, D), lambda i: (i, 0)),
        ],
        out_shape=[
            jax.ShapeDtypeStruct((N, 1), jnp.float32),
            jax.ShapeDtypeStruct((N, D), jnp.float32),
        ],
    )(x, W0, degp)


def _stats_body(accp_ref, hs_ref, dinv_ref, b_ref, comb_ref, s1_ref, s2_ref):
    i = pl.program_id(0)
    acc = accp_ref[0] + accp_ref[1]
    comb = (acc + hs_ref[...]) * dinv_ref[...] + b_ref[...]
    comb_ref[...] = comb
    p1 = jnp.sum(comb.reshape(_BR // 8, 8, D), axis=0)
    p2 = jnp.sum((comb * comb).reshape(_BR // 8, 8, D), axis=0)

    @pl.when(i == 0)
    def _():
        s1_ref[...] = p1
        s2_ref[...] = p2

    @pl.when(i > 0)
    def _():
        s1_ref[...] += p1
        s2_ref[...] += p2


def _tc_stats(accp, hs, dinv, b):
    return pl.pallas_call(
        _stats_body,
        grid=(_NB,),
        in_specs=[
            pl.BlockSpec((NC, _BR, D), lambda i: (0, i, 0)),
            pl.BlockSpec((_BR, D), lambda i: (i, 0)),
            pl.BlockSpec((_BR, 1), lambda i: (i, 0)),
            pl.BlockSpec((1, D), lambda i: (0, 0)),
        ],
        out_specs=[
            pl.BlockSpec((_BR, D), lambda i: (i, 0)),
            pl.BlockSpec((8, D), lambda i: (0, 0)),
            pl.BlockSpec((8, D), lambda i: (0, 0)),
        ],
        out_shape=[
            jax.ShapeDtypeStruct((N, D), jnp.float32),
            jax.ShapeDtypeStruct((8, D), jnp.float32),
            jax.ShapeDtypeStruct((8, D), jnp.float32),
        ],
    )(accp, hs, dinv, b.reshape(1, D))


def _bn_scale(s1_ref, s2_ref, g_ref, bt_ref):
    mu = jnp.sum(s1_ref[...], axis=0, keepdims=True) * (1.0 / N)
    ex2 = jnp.sum(s2_ref[...], axis=0, keepdims=True) * (1.0 / N)
    var = ex2 - mu * mu
    a = g_ref[...] * lax.rsqrt(var + 1e-5)
    c = bt_ref[...] - mu * a
    return a, c


def _next_body(comb_ref, s1_ref, s2_ref, g_ref, bt_ref, w_ref, dinv_ref, out_ref):
    a, c = _bn_scale(s1_ref, s2_ref, g_ref, bt_ref)
    h = jnp.maximum(comb_ref[...] * a + c, 0.0)
    out_ref[...] = jnp.dot(h, w_ref[...], preferred_element_type=jnp.float32) * dinv_ref[...]


def _tc_next(comb, s1, s2, g, bt, W, dinv):
    return pl.pallas_call(
        _next_body,
        grid=(_NB,),
        in_specs=[
            pl.BlockSpec((_BR, D), lambda i: (i, 0)),
            pl.BlockSpec((8, D), lambda i: (0, 0)),
            pl.BlockSpec((8, D), lambda i: (0, 0)),
            pl.BlockSpec((1, D), lambda i: (0, 0)),
            pl.BlockSpec((1, D), lambda i: (0, 0)),
            pl.BlockSpec((D, D), lambda i: (0, 0)),
            pl.BlockSpec((_BR, 1), lambda i: (i, 0)),
        ],
        out_specs=pl.BlockSpec((_BR, D), lambda i: (i, 0)),
        out_shape=jax.ShapeDtypeStruct((N, D), jnp.float32),
    )(comb, s1, s2, g.reshape(1, D), bt.reshape(1, D), W, dinv)


def _final_body(comb_ref, s1_ref, s2_ref, g_ref, bt_ref, wa_ref, wb_ref,
                outa_ref, outb_ref):
    a, c = _bn_scale(s1_ref, s2_ref, g_ref, bt_ref)
    h = jnp.maximum(comb_ref[...] * a + c, 0.0)
    outa_ref[...] = jnp.dot(h, wa_ref[...], preferred_element_type=jnp.float32)
    outb_ref[...] = jnp.dot(h, wb_ref[...], preferred_element_type=jnp.float32)


def _tc_final_nodes(comb, s1, s2, g, bt, WA, WB):
    return pl.pallas_call(
        _final_body,
        grid=(_NB,),
        in_specs=[
            pl.BlockSpec((_BR, D), lambda i: (i, 0)),
            pl.BlockSpec((8, D), lambda i: (0, 0)),
            pl.BlockSpec((8, D), lambda i: (0, 0)),
            pl.BlockSpec((1, D), lambda i: (0, 0)),
            pl.BlockSpec((1, D), lambda i: (0, 0)),
            pl.BlockSpec((D, D), lambda i: (0, 0)),
            pl.BlockSpec((D, D), lambda i: (0, 0)),
        ],
        out_specs=[
            pl.BlockSpec((_BR, D), lambda i: (i, 0)),
            pl.BlockSpec((_BR, D), lambda i: (i, 0)),
        ],
        out_shape=[
            jax.ShapeDtypeStruct((N, D), jnp.float32),
            jax.ShapeDtypeStruct((N, D), jnp.float32),
        ],
    )(comb, s1, s2, g.reshape(1, D), bt.reshape(1, D), WA, WB)


_EB = 2000
_NEB = E // _EB
_H2 = 64


def _mlp_body(g_ref, ea_ref, w1c_ref, b1_ref, w2_ref, b2_ref, w3_ref, b3_ref,
              out_ref):
    ea = ea_ref[...]
    ea = jnp.where(jnp.isnan(ea), 0.0, ea)
    z1 = g_ref[...] + jnp.dot(ea, w1c_ref[...], preferred_element_type=jnp.float32) + b1_ref[...]
    z1 = jnp.maximum(z1, 0.0)
    z2 = jnp.maximum(jnp.dot(z1, w2_ref[...], preferred_element_type=jnp.float32) + b2_ref[...], 0.0)
    out_ref[...] = jnp.dot(z2, w3_ref[...], preferred_element_type=jnp.float32) + b3_ref[...]


def _tc_mlp(gpairs, ea, W1c, bc1, Wc2, bc2, Wc3, bc3):
    return pl.pallas_call(
        _mlp_body,
        grid=(_NEB,),
        in_specs=[
            pl.BlockSpec((_EB, D), lambda i: (i, 0)),
            pl.BlockSpec((_EB, DE), lambda i: (i, 0)),
            pl.BlockSpec((DE, D), lambda i: (0, 0)),
            pl.BlockSpec((1, D), lambda i: (0, 0)),
            pl.BlockSpec((D, _H2), lambda i: (0, 0)),
            pl.BlockSpec((1, _H2), lambda i: (0, 0)),
            pl.BlockSpec((_H2, 2), lambda i: (0, 0)),
            pl.BlockSpec((1, 2), lambda i: (0, 0)),
        ],
        out_specs=pl.BlockSpec((_EB, 2), lambda i: (i, 0)),
        out_shape=jax.ShapeDtypeStruct((E, 2), jnp.float32),
    )(gpairs, ea, W1c, bc1.reshape(1, D), Wc2, bc2.reshape(1, _H2), Wc3,
      bc3.reshape(1, 2))


def kernel(x, edge_index, edge_attr, W0, b0, W1, b1, W2, b2, g0, bt0, g1, bt1,
           g2, bt2, Wc1, bc1, Wc2, bc2, Wc3, bc3):
    src = edge_index[0]
    dst = edge_index[1]
    pad = EPAD - E
    srcp = jnp.concatenate([src, jnp.zeros((pad,), jnp.int32)]).reshape(NW, NCH, CH)
    dstp_agg = jnp.concatenate([dst, jnp.full((pad,), TRASH, jnp.int32)]).reshape(NW, NCH, CH)
    dstp_pair = jnp.concatenate([dst, jnp.zeros((pad,), jnp.int32)]).reshape(NW, NCH, CH)

    degp = _sc_deg(dstp_agg)
    dinv, hs = _tc_prep(x, W0, degp)

    accp = _sc_agg(hs, srcp, dstp_agg)
    comb, s1, s2 = _tc_stats(accp, hs, dinv, b0)
    hs = _tc_next(comb, s1, s2, g0, bt0, W1, dinv)

    accp = _sc_agg(hs, srcp, dstp_agg)
    comb, s1, s2 = _tc_stats(accp, hs, dinv, b1)
    hs = _tc_next(comb, s1, s2, g1, bt1, W2, dinv)

    accp = _sc_agg(hs, srcp, dstp_agg)
    comb, s1, s2 = _tc_stats(accp, hs, dinv, b2)
    hA, hB = _tc_final_nodes(comb, s1, s2, g2, bt2, Wc1[:D], Wc1[D:2 * D])

    gpairs = _sc_pair(hA, hB, srcp, dstp_pair)
    return _tc_mlp(gpairs, edge_attr, Wc1[2 * D:], bc1, Wc2, bc2, Wc3, bc3)


# speedup vs baseline: 1.5706x; 1.5706x over previous
import functools

import jax
import jax.numpy as jnp
from jax import lax
from jax.experimental import pallas as pl
from jax.experimental.pallas import tpu as pltpu
from jax.experimental.pallas import tpu_sc as plsc

N = 10000
E = 320000
D = 128
DE = 16
NC = 2
NS = 16
NW = NC * NS
CH = 120
EPT = E // NW
NCH = 84
EPT_PAD = NCH * CH
EPAD = EPT_PAD * NW
TRASH = N
RPT = 632
NP = NS * RPT
ZCH = [(0, 120), (120, 120), (240, 120), (360, 120), (480, 120), (600, 32)]

_mesh = plsc.VectorSubcoreMesh(core_axis_name="c", subcore_axis_name="s",
                               num_cores=NC, num_subcores=NS)


def _zero_rows(buf, nrow, ncol):
    zeros16 = jnp.zeros((16,), jnp.float32)

    def body(r, _):
        for c in range(ncol // 16):
            buf[r, pl.ds(c * 16, 16)] = zeros16
        return 0

    lax.fori_loop(0, nrow, body, 0)


@functools.partial(
    pl.kernel,
    out_type=jax.ShapeDtypeStruct((NC, NP, D), jnp.float32),
    mesh=_mesh,
    scratch_types=[
        pltpu.VMEM((NCH, CH), jnp.int32),
        pltpu.VMEM((CH, D), jnp.float32),
        pltpu.VMEM_SHARED((NP, D), jnp.float32),
    ],
)
def _sc_deg(dstp, out, dst_v, val_v, acc_sh):
    cid = lax.axis_index("c")
    sid = lax.axis_index("s")
    wid = sid * NC + cid

    _zero_rows(val_v, CH, D)
    for off, nn in ZCH:
        pltpu.sync_copy(val_v.at[pl.ds(0, nn)],
                        acc_sh.at[pl.ds(sid * RPT + off, nn)])
    pltpu.sync_copy(dstp.at[wid], dst_v)
    plsc.subcore_barrier()

    ones16 = jnp.ones((16,), jnp.float32)

    def fill(r, _):
        for c in range(D // 16):
            val_v[r, pl.ds(c * 16, 16)] = ones16
        return 0

    lax.fori_loop(0, CH, fill, 0)

    def chunk(j, _):
        pltpu.sync_copy(val_v, acc_sh.at[dst_v.at[j]], add=True)
        return 0

    lax.fori_loop(0, NCH, chunk, 0)
    plsc.subcore_barrier()

    for off, nn in ZCH:
        sl = pl.ds(sid * RPT + off, nn)
        vv = val_v.at[pl.ds(0, nn)]
        pltpu.sync_copy(acc_sh.at[sl], vv)
        pltpu.sync_copy(vv, out.at[cid, sl])


@functools.partial(
    pl.kernel,
    out_type=jax.ShapeDtypeStruct((NC, NP, D), jnp.float32),
    mesh=_mesh,
    scratch_types=[
        [pltpu.VMEM((CH,), jnp.int32)] * 6,
        [pltpu.VMEM((CH,), jnp.int32)] * 6,
        [pltpu.VMEM((CH, D), jnp.float32)] * 3,
        [pltpu.SemaphoreType.DMA] * 6,
        [pltpu.SemaphoreType.DMA] * 3,
        pltpu.VMEM_SHARED((NP, D), jnp.float32),
    ],
)
def _sc_agg(table, srcp, dstp, out, sidx, didx, rows, si, sg, acc_sh):
    cid = lax.axis_index("c")
    sid = lax.axis_index("s")
    wid = sid * NC + cid

    _zero_rows(rows[0], CH, D)
    for off, nn in ZCH:
        pltpu.sync_copy(rows[0].at[pl.ds(0, nn)],
                        acc_sh.at[pl.ds(sid * RPT + off, nn)])
    plsc.subcore_barrier()

    pltpu.sync_copy(srcp.at[wid, 0], sidx[0])
    pltpu.sync_copy(dstp.at[wid, 0], didx[0])
    for k in range(1, 6):
        pltpu.async_copy(srcp.at[wid, k], sidx[k], si[k])
        pltpu.async_copy(dstp.at[wid, k], didx[k], si[k])
    pltpu.async_copy(table.at[sidx[0]], rows[0], sg[0])

    def chunk(j, _):
        m = lax.rem(j, 6)
        for k in range(6):
            kn = (k + 1) % 6
            rk = k % 3
            rn = (k + 1) % 3

            @pl.when(m == k)
            def _():
                @pl.when(j + 1 < NCH)
                def _():
                    pltpu.make_async_copy(srcp.at[wid, j + 1], sidx[kn], si[kn]).wait()
                    pltpu.make_async_copy(dstp.at[wid, j + 1], didx[kn], si[kn]).wait()
                    pltpu.async_copy(table.at[sidx[kn]], rows[rn], sg[rn])

                pltpu.make_async_copy(table.at[sidx[k]], rows[rk], sg[rk]).wait()
                pltpu.sync_copy(rows[rk], acc_sh.at[didx[k]], add=True)

                @pl.when(j + 6 < NCH)
                def _():
                    pltpu.async_copy(srcp.at[wid, j + 6], sidx[k], si[k])
                    pltpu.async_copy(dstp.at[wid, j + 6], didx[k], si[k])
        return 0

    lax.fori_loop(0, NCH, chunk, 0)
    plsc.subcore_barrier()

    for off, nn in ZCH:
        sl = pl.ds(sid * RPT + off, nn)
        vv = rows[0].at[pl.ds(0, nn)]
        pltpu.sync_copy(acc_sh.at[sl], vv)
        pltpu.sync_copy(vv, out.at[cid, sl])


@functools.partial(
    pl.kernel,
    out_type=jax.ShapeDtypeStruct((EPAD, D), jnp.float32),
    mesh=_mesh,
    scratch_types=[
        pltpu.VMEM((NCH, CH), jnp.int32),
        pltpu.VMEM((NCH, CH), jnp.int32),
        pltpu.VMEM((CH, D), jnp.float32),
        pltpu.VMEM((CH, D), jnp.float32),
        pltpu.VMEM((CH, D), jnp.float32),
        pltpu.VMEM((CH, D), jnp.float32),
        pltpu.SemaphoreType.DMA,
        pltpu.SemaphoreType.DMA,
        pltpu.SemaphoreType.DMA,
        pltpu.SemaphoreType.DMA,
    ],
)
def _sc_pair(hA, hB, srcp, dstp, out, src_v, dst_v, ra0, rb0, ra1, rb1,
             semA0, semB0, semA1, semB1):
    cid = lax.axis_index("c")
    sid = lax.axis_index("s")
    wid = sid * NC + cid

    pltpu.sync_copy(srcp.at[wid], src_v)
    pltpu.sync_copy(dstp.at[wid], dst_v)

    def add_store(ra, rb, j):
        def add_row(r, _):
            for c in range(D // 16):
                sl = pl.ds(c * 16, 16)
                ra[r, sl] = ra[r, sl] + rb[r, sl]
            return 0

        lax.fori_loop(0, CH, add_row, 0)
        pltpu.sync_copy(ra, out.at[pl.ds((wid * NCH + j) * CH, CH)])

    pltpu.async_copy(hA.at[src_v.at[0]], ra0, semA0)
    pltpu.async_copy(hB.at[dst_v.at[0]], rb0, semB0)

    def pair2(p, _):
        j0 = 2 * p
        j1 = j0 + 1
        pltpu.make_async_copy(hA.at[src_v.at[j0]], ra0, semA0).wait()
        pltpu.make_async_copy(hB.at[dst_v.at[j0]], rb0, semB0).wait()
        pltpu.async_copy(hA.at[src_v.at[j1]], ra1, semA1)
        pltpu.async_copy(hB.at[dst_v.at[j1]], rb1, semB1)
        add_store(ra0, rb0, j0)
        pltpu.make_async_copy(hA.at[src_v.at[j1]], ra1, semA1).wait()
        pltpu.make_async_copy(hB.at[dst_v.at[j1]], rb1, semB1).wait()

        @pl.when(p + 1 < NCH // 2)
        def _():
            pltpu.async_copy(hA.at[src_v.at[j0 + 2]], ra0, semA0)
            pltpu.async_copy(hB.at[dst_v.at[j0 + 2]], rb0, semB0)

        add_store(ra1, rb1, j1)
        return 0

    lax.fori_loop(0, NCH // 2, pair2, 0)


_NB = 10
_BR = N // _NB


def _prep_body(x_ref, w_ref, degp_ref, dinv_ref, hs_ref):
    deg = degp_ref[0, :, 0:1] + degp_ref[1, :, 0:1] + 1.0
    dinv = lax.rsqrt(deg)
    xv = x_ref[...]
    xv = jnp.where(jnp.isnan(xv), 0.0, xv)
    dinv_ref[...] = dinv
    hs_ref[...] = jnp.dot(xv, w_ref[...], preferred_element_type=jnp.float32) * dinv


def _tc_prep(x, W0, degp):
    return pl.pallas_call(
        _prep_body,
        grid=(_NB,),
        in_specs=[
            pl.BlockSpec((_BR, D), lambda i: (i, 0)),
            pl.BlockSpec((D, D), lambda i: (0, 0)),
            pl.BlockSpec((NC, _BR, D), lambda i: (0, i, 0)),
        ],
        out_specs=[
            pl.BlockSpec((_BR, 1), lambda i: (i, 0)),
            pl.BlockSpec((_BR, D), lambda i: (i, 0)),
        ],
        out_shape=[
            jax.ShapeDtypeStruct((N, 1), jnp.float32),
            jax.ShapeDtypeStruct((N, D), jnp.float32),
        ],
    )(x, W0, degp)


def _stats_body(accp_ref, hs_ref, dinv_ref, b_ref, comb_ref, s1_ref, s2_ref):
    i = pl.program_id(0)
    acc = accp_ref[0] + accp_ref[1]
    comb = (acc + hs_ref[...]) * dinv_ref[...] + b_ref[...]
    comb_ref[...] = comb
    p1 = jnp.sum(comb.reshape(_BR // 8, 8, D), axis=0)
    p2 = jnp.sum((comb * comb).reshape(_BR // 8, 8, D), axis=0)

    @pl.when(i == 0)
    def _():
        s1_ref[...] = p1
        s2_ref[...] = p2

    @pl.when(i > 0)
    def _():
        s1_ref[...] += p1
        s2_ref[...] += p2


def _tc_stats(accp, hs, dinv, b):
    return pl.pallas_call(
        _stats_body,
        grid=(_NB,),
        in_specs=[
            pl.BlockSpec((NC, _BR, D), lambda i: (0, i, 0)),
            pl.BlockSpec((_BR, D), lambda i: (i, 0)),
            pl.BlockSpec((_BR, 1), lambda i: (i, 0)),
            pl.BlockSpec((1, D), lambda i: (0, 0)),
        ],
        out_specs=[
            pl.BlockSpec((_BR, D), lambda i: (i, 0)),
            pl.BlockSpec((8, D), lambda i: (0, 0)),
            pl.BlockSpec((8, D), lambda i: (0, 0)),
        ],
        out_shape=[
            jax.ShapeDtypeStruct((N, D), jnp.float32),
            jax.ShapeDtypeStruct((8, D), jnp.float32),
            jax.ShapeDtypeStruct((8, D), jnp.float32),
        ],
    )(accp, hs, dinv, b.reshape(1, D))


def _bn_scale(s1_ref, s2_ref, g_ref, bt_ref):
    mu = jnp.sum(s1_ref[...], axis=0, keepdims=True) * (1.0 / N)
    ex2 = jnp.sum(s2_ref[...], axis=0, keepdims=True) * (1.0 / N)
    var = ex2 - mu * mu
    a = g_ref[...] * lax.rsqrt(var + 1e-5)
    c = bt_ref[...] - mu * a
    return a, c


def _next_body(comb_ref, s1_ref, s2_ref, g_ref, bt_ref, w_ref, dinv_ref, out_ref):
    a, c = _bn_scale(s1_ref, s2_ref, g_ref, bt_ref)
    h = jnp.maximum(comb_ref[...] * a + c, 0.0)
    out_ref[...] = jnp.dot(h, w_ref[...], preferred_element_type=jnp.float32) * dinv_ref[...]


def _tc_next(comb, s1, s2, g, bt, W, dinv):
    return pl.pallas_call(
        _next_body,
        grid=(_NB,),
        in_specs=[
            pl.BlockSpec((_BR, D), lambda i: (i, 0)),
            pl.BlockSpec((8, D), lambda i: (0, 0)),
            pl.BlockSpec((8, D), lambda i: (0, 0)),
            pl.BlockSpec((1, D), lambda i: (0, 0)),
            pl.BlockSpec((1, D), lambda i: (0, 0)),
            pl.BlockSpec((D, D), lambda i: (0, 0)),
            pl.BlockSpec((_BR, 1), lambda i: (i, 0)),
        ],
        out_specs=pl.BlockSpec((_BR, D), lambda i: (i, 0)),
        out_shape=jax.ShapeDtypeStruct((N, D), jnp.float32),
    )(comb, s1, s2, g.reshape(1, D), bt.reshape(1, D), W, dinv)


def _final_body(comb_ref, s1_ref, s2_ref, g_ref, bt_ref, wa_ref, wb_ref,
                outa_ref, outb_ref):
    a, c = _bn_scale(s1_ref, s2_ref, g_ref, bt_ref)
    h = jnp.maximum(comb_ref[...] * a + c, 0.0)
    outa_ref[...] = jnp.dot(h, wa_ref[...], preferred_element_type=jnp.float32)
    outb_ref[...] = jnp.dot(h, wb_ref[...], preferred_element_type=jnp.float32)


def _tc_final_nodes(comb, s1, s2, g, bt, WA, WB):
    return pl.pallas_call(
        _final_body,
        grid=(_NB,),
        in_specs=[
            pl.BlockSpec((_BR, D), lambda i: (i, 0)),
            pl.BlockSpec((8, D), lambda i: (0, 0)),
            pl.BlockSpec((8, D), lambda i: (0, 0)),
            pl.BlockSpec((1, D), lambda i: (0, 0)),
            pl.BlockSpec((1, D), lambda i: (0, 0)),
            pl.BlockSpec((D, D), lambda i: (0, 0)),
            pl.BlockSpec((D, D), lambda i: (0, 0)),
        ],
        out_specs=[
            pl.BlockSpec((_BR, D), lambda i: (i, 0)),
            pl.BlockSpec((_BR, D), lambda i: (i, 0)),
        ],
        out_shape=[
            jax.ShapeDtypeStruct((N, D), jnp.float32),
            jax.ShapeDtypeStruct((N, D), jnp.float32),
        ],
    )(comb, s1, s2, g.reshape(1, D), bt.reshape(1, D), WA, WB)


_EB = 2000
_NEB = E // _EB
_H2 = 64


def _mlp_body(g_ref, ea_ref, w1c_ref, b1_ref, w2_ref, b2_ref, w3_ref, b3_ref,
              out_ref):
    ea = ea_ref[...]
    ea = jnp.where(jnp.isnan(ea), 0.0, ea)
    z1 = g_ref[...] + jnp.dot(ea, w1c_ref[...], preferred_element_type=jnp.float32) + b1_ref[...]
    z1 = jnp.maximum(z1, 0.0)
    z2 = jnp.maximum(jnp.dot(z1, w2_ref[...], preferred_element_type=jnp.float32) + b2_ref[...], 0.0)
    out_ref[...] = jnp.dot(z2, w3_ref[...], preferred_element_type=jnp.float32) + b3_ref[...]


def _tc_mlp(gpairs, ea, W1c, bc1, Wc2, bc2, Wc3, bc3):
    return pl.pallas_call(
        _mlp_body,
        grid=(_NEB,),
        in_specs=[
            pl.BlockSpec((_EB, D), lambda i: (i, 0)),
            pl.BlockSpec((_EB, DE), lambda i: (i, 0)),
            pl.BlockSpec((DE, D), lambda i: (0, 0)),
            pl.BlockSpec((1, D), lambda i: (0, 0)),
            pl.BlockSpec((D, _H2), lambda i: (0, 0)),
            pl.BlockSpec((1, _H2), lambda i: (0, 0)),
            pl.BlockSpec((_H2, 2), lambda i: (0, 0)),
            pl.BlockSpec((1, 2), lambda i: (0, 0)),
        ],
        out_specs=pl.BlockSpec((_EB, 2), lambda i: (i, 0)),
        out_shape=jax.ShapeDtypeStruct((E, 2), jnp.float32),
    )(gpairs, ea, W1c, bc1.reshape(1, D), Wc2, bc2.reshape(1, _H2), Wc3,
      bc3.reshape(1, 2))


def kernel(x, edge_index, edge_attr, W0, b0, W1, b1, W2, b2, g0, bt0, g1, bt1,
           g2, bt2, Wc1, bc1, Wc2, bc2, Wc3, bc3):
    src = edge_index[0]
    dst = edge_index[1]
    pad = EPAD - E
    srcp = jnp.concatenate([src, jnp.zeros((pad,), jnp.int32)]).reshape(NW, NCH, CH)
    dstp_agg = jnp.concatenate([dst, jnp.full((pad,), TRASH, jnp.int32)]).reshape(NW, NCH, CH)
    dstp_pair = jnp.concatenate([dst, jnp.zeros((pad,), jnp.int32)]).reshape(NW, NCH, CH)

    degp = _sc_deg(dstp_agg)
    dinv, hs = _tc_prep(x, W0, degp)

    accp = _sc_agg(hs, srcp, dstp_agg)
    comb, s1, s2 = _tc_stats(accp, hs, dinv, b0)
    hs = _tc_next(comb, s1, s2, g0, bt0, W1, dinv)

    accp = _sc_agg(hs, srcp, dstp_agg)
    comb, s1, s2 = _tc_stats(accp, hs, dinv, b1)
    hs = _tc_next(comb, s1, s2, g1, bt1, W2, dinv)

    accp = _sc_agg(hs, srcp, dstp_agg)
    comb, s1, s2 = _tc_stats(accp, hs, dinv, b2)
    hA, hB = _tc_final_nodes(comb, s1, s2, g2, bt2, Wc1[:D], Wc1[D:2 * D])

    gpairs = _sc_pair(hA, hB, srcp, dstp_pair)
    return _tc_mlp(gpairs, edge_attr, Wc1[2 * D:], bc1, Wc2, bc2, Wc3, bc3)


# speedup vs baseline: 1.5991x; 1.0182x over previous
import functools

import jax
import jax.numpy as jnp
from jax import lax
from jax.experimental import pallas as pl
from jax.experimental.pallas import tpu as pltpu
from jax.experimental.pallas import tpu_sc as plsc

N = 10000
E = 320000
D = 128
DE = 16
NC = 2
NS = 16
NW = NC * NS
CH = 120
EPT = E // NW
NCH = 84
EPT_PAD = NCH * CH
EPAD = EPT_PAD * NW
TRASH = N
RPT = 632
NP = NS * RPT
ZCH = [(0, 120), (120, 120), (240, 120), (360, 120), (480, 120), (600, 32)]

_mesh = plsc.VectorSubcoreMesh(core_axis_name="c", subcore_axis_name="s",
                               num_cores=NC, num_subcores=NS)


def _zero_rows(buf, nrow, ncol):
    zeros16 = jnp.zeros((16,), jnp.float32)

    def body(r, _):
        for c in range(ncol // 16):
            buf[r, pl.ds(c * 16, 16)] = zeros16
        return 0

    lax.fori_loop(0, nrow, body, 0)


@functools.partial(
    pl.kernel,
    out_type=jax.ShapeDtypeStruct((NC, NP, D), jnp.float32),
    mesh=_mesh,
    scratch_types=[
        pltpu.VMEM((NCH, CH), jnp.int32),
        pltpu.VMEM((CH, D), jnp.float32),
        pltpu.VMEM_SHARED((NP, D), jnp.float32),
    ],
)
def _sc_deg(dstp, out, dst_v, val_v, acc_sh):
    cid = lax.axis_index("c")
    sid = lax.axis_index("s")
    wid = sid * NC + cid

    _zero_rows(val_v, CH, D)
    for off, nn in ZCH:
        pltpu.sync_copy(val_v.at[pl.ds(0, nn)],
                        acc_sh.at[pl.ds(sid * RPT + off, nn)])
    pltpu.sync_copy(dstp.at[wid], dst_v)
    plsc.subcore_barrier()

    ones16 = jnp.ones((16,), jnp.float32)

    def fill(r, _):
        for c in range(D // 16):
            val_v[r, pl.ds(c * 16, 16)] = ones16
        return 0

    lax.fori_loop(0, CH, fill, 0)

    def chunk(j, _):
        pltpu.sync_copy(val_v, acc_sh.at[dst_v.at[j]], add=True)
        return 0

    lax.fori_loop(0, NCH, chunk, 0)
    plsc.subcore_barrier()

    for off, nn in ZCH:
        sl = pl.ds(sid * RPT + off, nn)
        vv = val_v.at[pl.ds(0, nn)]
        pltpu.sync_copy(acc_sh.at[sl], vv)
        pltpu.sync_copy(vv, out.at[cid, sl])


@functools.partial(
    pl.kernel,
    out_type=jax.ShapeDtypeStruct((NC, NP, D), jnp.float32),
    mesh=_mesh,
    scratch_types=[
        [pltpu.VMEM((CH,), jnp.int32)] * 6,
        [pltpu.VMEM((CH,), jnp.int32)] * 6,
        [pltpu.VMEM((CH, D), jnp.float32)] * 3,
        [pltpu.SemaphoreType.DMA] * 6,
        [pltpu.SemaphoreType.DMA] * 3,
        pltpu.VMEM_SHARED((NP, D), jnp.float32),
    ],
)
def _sc_agg(table, srcp, dstp, out, sidx, didx, rows, si, sg, acc_sh):
    cid = lax.axis_index("c")
    sid = lax.axis_index("s")
    wid = sid * NC + cid

    _zero_rows(rows[0], CH, D)
    for off, nn in ZCH:
        pltpu.sync_copy(rows[0].at[pl.ds(0, nn)],
                        acc_sh.at[pl.ds(sid * RPT + off, nn)])
    plsc.subcore_barrier()

    pltpu.sync_copy(srcp.at[wid, 0], sidx[0])
    pltpu.sync_copy(dstp.at[wid, 0], didx[0])
    for k in range(1, 6):
        pltpu.async_copy(srcp.at[wid, k], sidx[k], si[k])
        pltpu.async_copy(dstp.at[wid, k], didx[k], si[k])
    pltpu.async_copy(table.at[sidx[0]], rows[0], sg[0])

    def chunk(j, _):
        m = lax.rem(j, 6)
        for k in range(6):
            kn = (k + 1) % 6
            rk = k % 3
            rn = (k + 1) % 3

            @pl.when(m == k)
            def _():
                @pl.when(j + 1 < NCH)
                def _():
                    pltpu.make_async_copy(srcp.at[wid, j + 1], sidx[kn], si[kn]).wait()
                    pltpu.make_async_copy(dstp.at[wid, j + 1], didx[kn], si[kn]).wait()
                    pltpu.async_copy(table.at[sidx[kn]], rows[rn], sg[rn])

                pltpu.make_async_copy(table.at[sidx[k]], rows[rk], sg[rk]).wait()
                pltpu.sync_copy(rows[rk], acc_sh.at[didx[k]], add=True)

                @pl.when(j + 6 < NCH)
                def _():
                    pltpu.async_copy(srcp.at[wid, j + 6], sidx[k], si[k])
                    pltpu.async_copy(dstp.at[wid, j + 6], didx[k], si[k])
        return 0

    lax.fori_loop(0, NCH, chunk, 0)
    plsc.subcore_barrier()

    for off, nn in ZCH:
        sl = pl.ds(sid * RPT + off, nn)
        vv = rows[0].at[pl.ds(0, nn)]
        pltpu.sync_copy(acc_sh.at[sl], vv)
        pltpu.sync_copy(vv, out.at[cid, sl])


@functools.partial(
    pl.kernel,
    out_type=jax.ShapeDtypeStruct((EPAD, D), jnp.float32),
    mesh=_mesh,
    scratch_types=[
        pltpu.VMEM((NCH, CH), jnp.int32),
        pltpu.VMEM((NCH, CH), jnp.int32),
        pltpu.VMEM((CH, D), jnp.float32),
        pltpu.VMEM((CH, D), jnp.float32),
        pltpu.VMEM((CH, D), jnp.float32),
        pltpu.VMEM((CH, D), jnp.float32),
        pltpu.SemaphoreType.DMA,
        pltpu.SemaphoreType.DMA,
        pltpu.SemaphoreType.DMA,
        pltpu.SemaphoreType.DMA,
    ],
)
def _sc_pair(hA, hB, srcp, dstp, out, src_v, dst_v, ra0, rb0, ra1, rb1,
             semA0, semB0, semA1, semB1):
    cid = lax.axis_index("c")
    sid = lax.axis_index("s")
    wid = sid * NC + cid

    pltpu.sync_copy(srcp.at[wid], src_v)
    pltpu.sync_copy(dstp.at[wid], dst_v)

    def add_store(ra, rb, j):
        def add_row(r, _):
            for c in range(D // 16):
                sl = pl.ds(c * 16, 16)
                ra[r, sl] = ra[r, sl] + rb[r, sl]
            return 0

        lax.fori_loop(0, CH, add_row, 0)
        pltpu.sync_copy(ra, out.at[pl.ds((wid * NCH + j) * CH, CH)])

    pltpu.async_copy(hA.at[src_v.at[0]], ra0, semA0)
    pltpu.async_copy(hB.at[dst_v.at[0]], rb0, semB0)

    def pair2(p, _):
        j0 = 2 * p
        j1 = j0 + 1
        pltpu.make_async_copy(hA.at[src_v.at[j0]], ra0, semA0).wait()
        pltpu.make_async_copy(hB.at[dst_v.at[j0]], rb0, semB0).wait()
        pltpu.async_copy(hA.at[src_v.at[j1]], ra1, semA1)
        pltpu.async_copy(hB.at[dst_v.at[j1]], rb1, semB1)
        add_store(ra0, rb0, j0)
        pltpu.make_async_copy(hA.at[src_v.at[j1]], ra1, semA1).wait()
        pltpu.make_async_copy(hB.at[dst_v.at[j1]], rb1, semB1).wait()

        @pl.when(p + 1 < NCH // 2)
        def _():
            pltpu.async_copy(hA.at[src_v.at[j0 + 2]], ra0, semA0)
            pltpu.async_copy(hB.at[dst_v.at[j0 + 2]], rb0, semB0)

        add_store(ra1, rb1, j1)
        return 0

    lax.fori_loop(0, NCH // 2, pair2, 0)


_NB = 10
_BR = N // _NB


def _mm0_body(x_ref, w_ref, hw_ref):
    xv = x_ref[...]
    xv = jnp.where(jnp.isnan(xv), 0.0, xv)
    hw_ref[...] = jnp.dot(xv, w_ref[...], preferred_element_type=jnp.float32)


def _tc_mm0(x, W0):
    return pl.pallas_call(
        _mm0_body,
        grid=(_NB,),
        in_specs=[
            pl.BlockSpec((_BR, D), lambda i: (i, 0)),
            pl.BlockSpec((D, D), lambda i: (0, 0)),
        ],
        out_specs=pl.BlockSpec((_BR, D), lambda i: (i, 0)),
        out_shape=jax.ShapeDtypeStruct((N, D), jnp.float32),
    )(x, W0)


def _scale_body(hw_ref, degp_ref, dinv_ref, hs_ref):
    deg = degp_ref[0, :, 0:1] + degp_ref[1, :, 0:1] + 1.0
    dinv = lax.rsqrt(deg)
    dinv_ref[...] = dinv
    hs_ref[...] = hw_ref[...] * dinv


def _tc_prep(hw, degp):
    return pl.pallas_call(
        _scale_body,
        grid=(_NB,),
        in_specs=[
            pl.BlockSpec((_BR, D), lambda i: (i, 0)),
            pl.BlockSpec((NC, _BR, D), lambda i: (0, i, 0)),
        ],
        out_specs=[
            pl.BlockSpec((_BR, 1), lambda i: (i, 0)),
            pl.BlockSpec((_BR, D), lambda i: (i, 0)),
        ],
        out_shape=[
            jax.ShapeDtypeStruct((N, 1), jnp.float32),
            jax.ShapeDtypeStruct((N, D), jnp.float32),
        ],
    )(hw, degp)


def _stats_body(accp_ref, hs_ref, dinv_ref, b_ref, comb_ref, s1_ref, s2_ref):
    i = pl.program_id(0)
    acc = accp_ref[0] + accp_ref[1]
    comb = (acc + hs_ref[...]) * dinv_ref[...] + b_ref[...]
    comb_ref[...] = comb
    p1 = jnp.sum(comb.reshape(_BR // 8, 8, D), axis=0)
    p2 = jnp.sum((comb * comb).reshape(_BR // 8, 8, D), axis=0)

    @pl.when(i == 0)
    def _():
        s1_ref[...] = p1
        s2_ref[...] = p2

    @pl.when(i > 0)
    def _():
        s1_ref[...] += p1
        s2_ref[...] += p2


def _tc_stats(accp, hs, dinv, b):
    return pl.pallas_call(
        _stats_body,
        grid=(_NB,),
        in_specs=[
            pl.BlockSpec((NC, _BR, D), lambda i: (0, i, 0)),
            pl.BlockSpec((_BR, D), lambda i: (i, 0)),
            pl.BlockSpec((_BR, 1), lambda i: (i, 0)),
            pl.BlockSpec((1, D), lambda i: (0, 0)),
        ],
        out_specs=[
            pl.BlockSpec((_BR, D), lambda i: (i, 0)),
            pl.BlockSpec((8, D), lambda i: (0, 0)),
            pl.BlockSpec((8, D), lambda i: (0, 0)),
        ],
        out_shape=[
            jax.ShapeDtypeStruct((N, D), jnp.float32),
            jax.ShapeDtypeStruct((8, D), jnp.float32),
            jax.ShapeDtypeStruct((8, D), jnp.float32),
        ],
    )(accp, hs, dinv, b.reshape(1, D))


def _bn_scale(s1_ref, s2_ref, g_ref, bt_ref):
    mu = jnp.sum(s1_ref[...], axis=0, keepdims=True) * (1.0 / N)
    ex2 = jnp.sum(s2_ref[...], axis=0, keepdims=True) * (1.0 / N)
    var = ex2 - mu * mu
    a = g_ref[...] * lax.rsqrt(var + 1e-5)
    c = bt_ref[...] - mu * a
    return a, c


def _next_body(comb_ref, s1_ref, s2_ref, g_ref, bt_ref, w_ref, dinv_ref, out_ref):
    a, c = _bn_scale(s1_ref, s2_ref, g_ref, bt_ref)
    h = jnp.maximum(comb_ref[...] * a + c, 0.0)
    out_ref[...] = jnp.dot(h, w_ref[...], preferred_element_type=jnp.float32) * dinv_ref[...]


def _tc_next(comb, s1, s2, g, bt, W, dinv):
    return pl.pallas_call(
        _next_body,
        grid=(_NB,),
        in_specs=[
            pl.BlockSpec((_BR, D), lambda i: (i, 0)),
            pl.BlockSpec((8, D), lambda i: (0, 0)),
            pl.BlockSpec((8, D), lambda i: (0, 0)),
            pl.BlockSpec((1, D), lambda i: (0, 0)),
            pl.BlockSpec((1, D), lambda i: (0, 0)),
            pl.BlockSpec((D, D), lambda i: (0, 0)),
            pl.BlockSpec((_BR, 1), lambda i: (i, 0)),
        ],
        out_specs=pl.BlockSpec((_BR, D), lambda i: (i, 0)),
        out_shape=jax.ShapeDtypeStruct((N, D), jnp.float32),
    )(comb, s1, s2, g.reshape(1, D), bt.reshape(1, D), W, dinv)


def _final_body(comb_ref, s1_ref, s2_ref, g_ref, bt_ref, wa_ref, wb_ref,
                outa_ref, outb_ref):
    a, c = _bn_scale(s1_ref, s2_ref, g_ref, bt_ref)
    h = jnp.maximum(comb_ref[...] * a + c, 0.0)
    outa_ref[...] = jnp.dot(h, wa_ref[...], preferred_element_type=jnp.float32)
    outb_ref[...] = jnp.dot(h, wb_ref[...], preferred_element_type=jnp.float32)


def _tc_final_nodes(comb, s1, s2, g, bt, WA, WB):
    return pl.pallas_call(
        _final_body,
        grid=(_NB,),
        in_specs=[
            pl.BlockSpec((_BR, D), lambda i: (i, 0)),
            pl.BlockSpec((8, D), lambda i: (0, 0)),
            pl.BlockSpec((8, D), lambda i: (0, 0)),
            pl.BlockSpec((1, D), lambda i: (0, 0)),
            pl.BlockSpec((1, D), lambda i: (0, 0)),
            pl.BlockSpec((D, D), lambda i: (0, 0)),
            pl.BlockSpec((D, D), lambda i: (0, 0)),
        ],
        out_specs=[
            pl.BlockSpec((_BR, D), lambda i: (i, 0)),
            pl.BlockSpec((_BR, D), lambda i: (i, 0)),
        ],
        out_shape=[
            jax.ShapeDtypeStruct((N, D), jnp.float32),
            jax.ShapeDtypeStruct((N, D), jnp.float32),
        ],
    )(comb, s1, s2, g.reshape(1, D), bt.reshape(1, D), WA, WB)


_EB = 2000
_NEB = E // _EB
_H2 = 64


def _mlp_body(g_ref, ea_ref, w1c_ref, b1_ref, w2_ref, b2_ref, w3_ref, b3_ref,
              out_ref):
    ea = ea_ref[...]
    ea = jnp.where(jnp.isnan(ea), 0.0, ea)
    z1 = g_ref[...] + jnp.dot(ea, w1c_ref[...], preferred_element_type=jnp.float32) + b1_ref[...]
    z1 = jnp.maximum(z1, 0.0)
    z2 = jnp.maximum(jnp.dot(z1, w2_ref[...], preferred_element_type=jnp.float32) + b2_ref[...], 0.0)
    out_ref[...] = jnp.dot(z2, w3_ref[...], preferred_element_type=jnp.float32) + b3_ref[...]


def _tc_mlp(gpairs, ea, W1c, bc1, Wc2, bc2, Wc3, bc3):
    return pl.pallas_call(
        _mlp_body,
        grid=(_NEB,),
        in_specs=[
            pl.BlockSpec((_EB, D), lambda i: (i, 0)),
            pl.BlockSpec((_EB, DE), lambda i: (i, 0)),
            pl.BlockSpec((DE, D), lambda i: (0, 0)),
            pl.BlockSpec((1, D), lambda i: (0, 0)),
            pl.BlockSpec((D, _H2), lambda i: (0, 0)),
            pl.BlockSpec((1, _H2), lambda i: (0, 0)),
            pl.BlockSpec((_H2, 2), lambda i: (0, 0)),
            pl.BlockSpec((1, 2), lambda i: (0, 0)),
        ],
        out_specs=pl.BlockSpec((_EB, 2), lambda i: (i, 0)),
        out_shape=jax.ShapeDtypeStruct((E, 2), jnp.float32),
    )(gpairs, ea, W1c, bc1.reshape(1, D), Wc2, bc2.reshape(1, _H2), Wc3,
      bc3.reshape(1, 2))


def kernel(x, edge_index, edge_attr, W0, b0, W1, b1, W2, b2, g0, bt0, g1, bt1,
           g2, bt2, Wc1, bc1, Wc2, bc2, Wc3, bc3):
    src = edge_index[0]
    dst = edge_index[1]
    pad = EPAD - E
    srcp = jnp.concatenate([src, jnp.zeros((pad,), jnp.int32)]).reshape(NW, NCH, CH)
    dstp_agg = jnp.concatenate([dst, jnp.full((pad,), TRASH, jnp.int32)]).reshape(NW, NCH, CH)
    dstp_pair = jnp.concatenate([dst, jnp.zeros((pad,), jnp.int32)]).reshape(NW, NCH, CH)

    hw0 = _tc_mm0(x, W0)
    degp = _sc_deg(dstp_agg)
    dinv, hs = _tc_prep(hw0, degp)

    accp = _sc_agg(hs, srcp, dstp_agg)
    comb, s1, s2 = _tc_stats(accp, hs, dinv, b0)
    hs = _tc_next(comb, s1, s2, g0, bt0, W1, dinv)

    accp = _sc_agg(hs, srcp, dstp_agg)
    comb, s1, s2 = _tc_stats(accp, hs, dinv, b1)
    hs = _tc_next(comb, s1, s2, g1, bt1, W2, dinv)

    accp = _sc_agg(hs, srcp, dstp_agg)
    comb, s1, s2 = _tc_stats(accp, hs, dinv, b2)
    hA, hB = _tc_final_nodes(comb, s1, s2, g2, bt2, Wc1[:D], Wc1[D:2 * D])

    gpairs = _sc_pair(hA, hB, srcp, dstp_pair)
    return _tc_mlp(gpairs, edge_attr, Wc1[2 * D:], bc1, Wc2, bc2, Wc3, bc3)
